# baseline (speedup 1.0000x reference)
import functools

import jax
import jax.numpy as jnp
from jax import lax
from jax.experimental import pallas as pl
from jax.experimental.pallas import tpu as pltpu
from jax.experimental.pallas import tpu_sc as plsc

_B = 4096
_N_SO3 = 4608
_N_PSI = 24
_N_CONES = 192

_ROW_BLK = 1024
_CHUNK1 = 3072

_NC = 1
_NS = 16
_NW = _NC * _NS
_LANES = 16


def _tc_cone_body(grid_ref, pred_ref, out_ref):
    sim_t = lax.dot_general(
        grid_ref[...],
        pred_ref[...],
        dimension_numbers=(((1,), (1,)), ((), ())),
        preferred_element_type=jnp.float32,
    )
    cmax = jnp.max(sim_t.reshape(_N_CONES, _N_PSI, _ROW_BLK), axis=1)
    m = jnp.max(cmax, axis=0, keepdims=True)
    row = lax.broadcasted_iota(jnp.int32, (_N_CONES, _ROW_BLK), 0)
    out_ref[...] = jnp.min(jnp.where(cmax == m, row, _N_CONES), axis=0)


def _tc_cone_indices(pred9, grid9):
    n = pred9.shape[0]
    return pl.pallas_call(
        _tc_cone_body,
        grid=(n // _ROW_BLK,),
        in_specs=[
            pl.BlockSpec((_N_SO3, 9), lambda i: (0, 0)),
            pl.BlockSpec((_ROW_BLK, 9), lambda i: (i, 0)),
        ],
        out_specs=pl.BlockSpec((_ROW_BLK,), lambda i: (i,)),
        out_shape=jax.ShapeDtypeStruct((n,), jnp.int32),
    )(grid9, pred9)


_N_TABV = _N_CONES // _LANES


def _vgather(vec, lane):
    return vec.at[lane].get(mode="promise_in_bounds")


def _make_sc_body(per_w):
    def body(cone_hbm, scores_hbm, med_hbm, mad_hbm, out_hbm,
             idx_v, s_v, medt_v, madt_v, o_v, sem1, sem2, sem3, sem4):
        wid = lax.axis_index("s") * _NC + lax.axis_index("c")
        base = wid * per_w
        c_idx = pltpu.async_copy(cone_hbm.at[pl.ds(base, per_w)], idx_v, sem1)
        c_s = pltpu.async_copy(scores_hbm.at[pl.ds(base, per_w)], s_v, sem2)
        c_med = pltpu.async_copy(med_hbm, medt_v, sem3)
        c_mad = pltpu.async_copy(mad_hbm, madt_v, sem4)
        c_idx.wait()
        c_s.wait()
        c_med.wait()
        c_mad.wait()
        medt = [medt_v[pl.ds(t * _LANES, _LANES)] for t in range(_N_TABV)]
        madt = [madt_v[pl.ds(t * _LANES, _LANES)] for t in range(_N_TABV)]
        for i in range(per_w // _LANES):
            sl = pl.ds(i * _LANES, _LANES)
            idx = idx_v[sl]
            grp = lax.shift_right_logical(idx, 4)
            lane = lax.bitwise_and(idx, 15)
            med = _vgather(medt[0], lane)
            mad = _vgather(madt[0], lane)
            for t in range(1, _N_TABV):
                sel = grp == t
                med = jnp.where(sel, _vgather(medt[t], lane), med)
                mad = jnp.where(sel, _vgather(madt[t], lane), mad)
            o_v[sl] = (s_v[sl] - med) / mad
        pltpu.sync_copy(o_v, out_hbm.at[pl.ds(base, per_w)])
    return body


@functools.cache
def _sc_normalize(n):
    per_w = n // _NW
    mesh = plsc.VectorSubcoreMesh(
        core_axis_name="c", subcore_axis_name="s", num_cores=_NC)
    return pl.kernel(
        _make_sc_body(per_w),
        mesh=mesh,
        out_type=jax.ShapeDtypeStruct((n,), jnp.float32),
        scratch_types=[
            pltpu.VMEM((per_w,), jnp.int32),
            pltpu.VMEM((per_w,), jnp.float32),
            pltpu.VMEM((_N_CONES,), jnp.float32),
            pltpu.VMEM((_N_CONES,), jnp.float32),
            pltpu.VMEM((per_w,), jnp.float32),
            pltpu.SemaphoreType.DMA,
            pltpu.SemaphoreType.DMA,
            pltpu.SemaphoreType.DMA,
            pltpu.SemaphoreType.DMA,
        ],
    )


def kernel(pred_rotmats, scores, grid_rotmats, medians, mads):
    c1, c2 = _CHUNK1, _B - _CHUNK1
    grid9 = grid_rotmats.reshape(_N_SO3, 9)
    pred9 = pred_rotmats.reshape(_B, 9)
    cone1 = _tc_cone_indices(pred9[:c1], grid9)
    cone2 = _tc_cone_indices(pred9[c1:], grid9)
    out1 = _sc_normalize(c1)(cone1, scores[:c1], medians, mads)
    out2 = _sc_normalize(c2)(cone2, scores[c1:], medians, mads)
    return jnp.concatenate([out1, out2])


# speedup vs baseline: 1.0069x; 1.0069x over previous
import functools

import jax
import jax.numpy as jnp
from jax import lax
from jax.experimental import pallas as pl
from jax.experimental.pallas import tpu as pltpu
from jax.experimental.pallas import tpu_sc as plsc

_B = 4096
_N_SO3 = 4608
_N_PSI = 24
_N_CONES = 192

_ROW_BLK = 1024
_CHUNK1 = 2048

_NC = 1
_NS = 16
_NW = _NC * _NS
_LANES = 16


def _tc_cone_body(grid_ref, pred_ref, out_ref):
    sim_t = lax.dot_general(
        grid_ref[...],
        pred_ref[...],
        dimension_numbers=(((1,), (1,)), ((), ())),
        preferred_element_type=jnp.float32,
    )
    cmax = jnp.max(sim_t.reshape(_N_CONES, _N_PSI, _ROW_BLK), axis=1)
    m = jnp.max(cmax, axis=0, keepdims=True)
    row = lax.broadcasted_iota(jnp.int32, (_N_CONES, _ROW_BLK), 0)
    out_ref[...] = jnp.min(jnp.where(cmax == m, row, _N_CONES), axis=0)


def _tc_cone_indices(pred9, grid9):
    n = pred9.shape[0]
    return pl.pallas_call(
        _tc_cone_body,
        grid=(n // _ROW_BLK,),
        in_specs=[
            pl.BlockSpec((_N_SO3, 9), lambda i: (0, 0)),
            pl.BlockSpec((_ROW_BLK, 9), lambda i: (i, 0)),
        ],
        out_specs=pl.BlockSpec((_ROW_BLK,), lambda i: (i,)),
        out_shape=jax.ShapeDtypeStruct((n,), jnp.int32),
    )(grid9, pred9)


_N_TABV = _N_CONES // _LANES


def _vgather(vec, lane):
    return vec.at[lane].get(mode="promise_in_bounds")


def _make_sc_body(per_w):
    def body(cone_hbm, scores_hbm, med_hbm, mad_hbm, out_hbm,
             idx_v, s_v, medt_v, madt_v, o_v, sem1, sem2, sem3, sem4):
        wid = lax.axis_index("s") * _NC + lax.axis_index("c")
        base = wid * per_w
        c_idx = pltpu.async_copy(cone_hbm.at[pl.ds(base, per_w)], idx_v, sem1)
        c_s = pltpu.async_copy(scores_hbm.at[pl.ds(base, per_w)], s_v, sem2)
        c_med = pltpu.async_copy(med_hbm, medt_v, sem3)
        c_mad = pltpu.async_copy(mad_hbm, madt_v, sem4)
        c_idx.wait()
        c_s.wait()
        c_med.wait()
        c_mad.wait()
        medt = [medt_v[pl.ds(t * _LANES, _LANES)] for t in range(_N_TABV)]
        madt = [madt_v[pl.ds(t * _LANES, _LANES)] for t in range(_N_TABV)]
        for i in range(per_w // _LANES):
            sl = pl.ds(i * _LANES, _LANES)
            idx = idx_v[sl]
            grp = lax.shift_right_logical(idx, 4)
            lane = lax.bitwise_and(idx, 15)
            med = _vgather(medt[0], lane)
            mad = _vgather(madt[0], lane)
            for t in range(1, _N_TABV):
                sel = grp == t
                med = jnp.where(sel, _vgather(medt[t], lane), med)
                mad = jnp.where(sel, _vgather(madt[t], lane), mad)
            o_v[sl] = (s_v[sl] - med) / mad
        pltpu.sync_copy(o_v, out_hbm.at[pl.ds(base, per_w)])
    return body


@functools.cache
def _sc_normalize(n):
    per_w = n // _NW
    mesh = plsc.VectorSubcoreMesh(
        core_axis_name="c", subcore_axis_name="s", num_cores=_NC)
    return pl.kernel(
        _make_sc_body(per_w),
        mesh=mesh,
        out_type=jax.ShapeDtypeStruct((n,), jnp.float32),
        scratch_types=[
            pltpu.VMEM((per_w,), jnp.int32),
            pltpu.VMEM((per_w,), jnp.float32),
            pltpu.VMEM((_N_CONES,), jnp.float32),
            pltpu.VMEM((_N_CONES,), jnp.float32),
            pltpu.VMEM((per_w,), jnp.float32),
            pltpu.SemaphoreType.DMA,
            pltpu.SemaphoreType.DMA,
            pltpu.SemaphoreType.DMA,
            pltpu.SemaphoreType.DMA,
        ],
    )


def kernel(pred_rotmats, scores, grid_rotmats, medians, mads):
    c1, c2 = _CHUNK1, _B - _CHUNK1
    grid9 = grid_rotmats.reshape(_N_SO3, 9)
    pred9 = pred_rotmats.reshape(_B, 9)
    cone1 = _tc_cone_indices(pred9[:c1], grid9)
    cone2 = _tc_cone_indices(pred9[c1:], grid9)
    out1 = _sc_normalize(c1)(cone1, scores[:c1], medians, mads)
    out2 = _sc_normalize(c2)(cone2, scores[c1:], medians, mads)
    return jnp.concatenate([out1, out2])


# speedup vs baseline: 1.1375x; 1.1297x over previous
import functools

import jax
import jax.numpy as jnp
from jax import lax
from jax.experimental import pallas as pl
from jax.experimental.pallas import tpu as pltpu
from jax.experimental.pallas import tpu_sc as plsc

_B = 4096
_N_SO3 = 4608
_N_PSI = 24
_N_CONES = 192

_ROW_BLK = 1024
_CHUNK1 = 2048

_NC = 1
_NS = 16
_NW = _NC * _NS
_LANES = 16


def _tc_cone_body(grid_ref, pred_ref, out_ref):
    sim_t = lax.dot_general(
        grid_ref[...],
        pred_ref[...],
        dimension_numbers=(((1,), (1,)), ((), ())),
        preferred_element_type=jnp.float32,
    )
    cmax = jnp.max(sim_t.reshape(_N_CONES, _N_PSI, _ROW_BLK), axis=1)
    m = jnp.max(cmax, axis=0, keepdims=True)
    row = lax.broadcasted_iota(jnp.int32, (_N_CONES, _ROW_BLK), 0)
    out_ref[...] = jnp.min(jnp.where(cmax == m, row, _N_CONES), axis=0)


def _tc_cone_indices(pred9, grid9):
    n = pred9.shape[0]
    return pl.pallas_call(
        _tc_cone_body,
        grid=(n // _ROW_BLK,),
        in_specs=[
            pl.BlockSpec((_N_SO3, 9), lambda i: (0, 0)),
            pl.BlockSpec((_ROW_BLK, 9), lambda i: (i, 0)),
        ],
        out_specs=pl.BlockSpec((_ROW_BLK,), lambda i: (i,)),
        out_shape=jax.ShapeDtypeStruct((n,), jnp.int32),
    )(grid9, pred9)


_N_TABV = _N_CONES // _LANES


def _vgather(vec, lane):
    return vec.at[lane].get(mode="promise_in_bounds")


def _make_sc_body(per_w):
    def body(cone_hbm, scores_hbm, med_hbm, mad_hbm, out_hbm,
             idx_v, s_v, medt_v, madt_v, o_v, sem1, sem2, sem3, sem4):
        wid = lax.axis_index("s") * _NC + lax.axis_index("c")
        base = wid * per_w
        c_idx = pltpu.async_copy(cone_hbm.at[pl.ds(base, per_w)], idx_v, sem1)
        c_s = pltpu.async_copy(scores_hbm.at[pl.ds(base, per_w)], s_v, sem2)
        c_med = pltpu.async_copy(med_hbm, medt_v, sem3)
        c_mad = pltpu.async_copy(mad_hbm, madt_v, sem4)
        c_idx.wait()
        c_s.wait()
        c_med.wait()
        c_mad.wait()
        medt = [medt_v[pl.ds(t * _LANES, _LANES)] for t in range(_N_TABV)]
        madt = [madt_v[pl.ds(t * _LANES, _LANES)] for t in range(_N_TABV)]
        for i in range(per_w // _LANES):
            sl = pl.ds(i * _LANES, _LANES)
            idx = idx_v[sl]
            grp = lax.shift_right_logical(idx, 4)
            lane = lax.bitwise_and(idx, 15)
            med = _vgather(medt[0], lane)
            mad = _vgather(madt[0], lane)
            for t in range(1, _N_TABV):
                sel = grp == t
                med = jnp.where(sel, _vgather(medt[t], lane), med)
                mad = jnp.where(sel, _vgather(madt[t], lane), mad)
            o_v[sl] = (s_v[sl] - med) / mad
        pltpu.sync_copy(o_v, out_hbm.at[pl.ds(base, per_w)])
    return body


@functools.cache
def _sc_normalize(n):
    per_w = n // _NW
    mesh = plsc.VectorSubcoreMesh(
        core_axis_name="c", subcore_axis_name="s", num_cores=_NC)
    return pl.kernel(
        _make_sc_body(per_w),
        mesh=mesh,
        out_type=jax.ShapeDtypeStruct((n,), jnp.float32),
        scratch_types=[
            pltpu.VMEM((per_w,), jnp.int32),
            pltpu.VMEM((per_w,), jnp.float32),
            pltpu.VMEM((_N_CONES,), jnp.float32),
            pltpu.VMEM((_N_CONES,), jnp.float32),
            pltpu.VMEM((per_w,), jnp.float32),
            pltpu.SemaphoreType.DMA,
            pltpu.SemaphoreType.DMA,
            pltpu.SemaphoreType.DMA,
            pltpu.SemaphoreType.DMA,
        ],
    )


def kernel(pred_rotmats, scores, grid_rotmats, medians, mads):
    grid9 = grid_rotmats.reshape(_N_SO3, 9)
    pred9 = pred_rotmats.reshape(_B, 9)
    cone = _tc_cone_indices(pred9, grid9)
    return _sc_normalize(_B)(cone, scores, medians, mads)


# speedup vs baseline: 1.1698x; 1.0284x over previous
import functools

import jax
import jax.numpy as jnp
from jax import lax
from jax.experimental import pallas as pl
from jax.experimental.pallas import tpu as pltpu
from jax.experimental.pallas import tpu_sc as plsc

_B = 4096
_N_SO3 = 4608
_N_PSI = 24
_N_CONES = 192

_ROW_BLK = 1024
_CHUNK1 = 2048

_NC = 1
_NS = 16
_NW = _NC * _NS
_LANES = 16


def _tc_cone_body(grid_ref, pred_ref, out_ref):
    sim_t = lax.dot_general(
        grid_ref[...],
        pred_ref[...],
        dimension_numbers=(((1,), (1,)), ((), ())),
        preferred_element_type=jnp.float32,
    )
    cmax = jnp.max(sim_t.reshape(_N_CONES, _N_PSI, _ROW_BLK), axis=1)
    m = jnp.max(cmax, axis=0, keepdims=True)
    row = lax.broadcasted_iota(jnp.int32, (_N_CONES, _ROW_BLK), 0)
    pick = lax.bitcast_convert_type(
        jnp.where(cmax == m, row, _N_CONES), jnp.float32)
    out_ref[...] = lax.bitcast_convert_type(jnp.min(pick, axis=0), jnp.int32)


def _tc_cone_indices(pred9, grid9):
    n = pred9.shape[0]
    return pl.pallas_call(
        _tc_cone_body,
        grid=(n // _ROW_BLK,),
        in_specs=[
            pl.BlockSpec((_N_SO3, 9), lambda i: (0, 0)),
            pl.BlockSpec((_ROW_BLK, 9), lambda i: (i, 0)),
        ],
        out_specs=pl.BlockSpec((_ROW_BLK,), lambda i: (i,)),
        out_shape=jax.ShapeDtypeStruct((n,), jnp.int32),
    )(grid9, pred9)


_N_TABV = _N_CONES // _LANES


def _vgather(vec, lane):
    return vec.at[lane].get(mode="promise_in_bounds")


def _make_sc_body(per_w):
    def body(cone_hbm, scores_hbm, med_hbm, mad_hbm, out_hbm,
             idx_v, s_v, medt_v, madt_v, o_v, sem1, sem2, sem3, sem4):
        wid = lax.axis_index("s") * _NC + lax.axis_index("c")
        base = wid * per_w
        c_idx = pltpu.async_copy(cone_hbm.at[pl.ds(base, per_w)], idx_v, sem1)
        c_s = pltpu.async_copy(scores_hbm.at[pl.ds(base, per_w)], s_v, sem2)
        c_med = pltpu.async_copy(med_hbm, medt_v, sem3)
        c_mad = pltpu.async_copy(mad_hbm, madt_v, sem4)
        c_idx.wait()
        c_s.wait()
        c_med.wait()
        c_mad.wait()
        medt = [medt_v[pl.ds(t * _LANES, _LANES)] for t in range(_N_TABV)]
        madt = [madt_v[pl.ds(t * _LANES, _LANES)] for t in range(_N_TABV)]
        for i in range(per_w // _LANES):
            sl = pl.ds(i * _LANES, _LANES)
            idx = idx_v[sl]
            grp = lax.shift_right_logical(idx, 4)
            lane = lax.bitwise_and(idx, 15)
            med = _vgather(medt[0], lane)
            mad = _vgather(madt[0], lane)
            for t in range(1, _N_TABV):
                sel = grp == t
                med = jnp.where(sel, _vgather(medt[t], lane), med)
                mad = jnp.where(sel, _vgather(madt[t], lane), mad)
            o_v[sl] = (s_v[sl] - med) / mad
        pltpu.sync_copy(o_v, out_hbm.at[pl.ds(base, per_w)])
    return body


@functools.cache
def _sc_normalize(n):
    per_w = n // _NW
    mesh = plsc.VectorSubcoreMesh(
        core_axis_name="c", subcore_axis_name="s", num_cores=_NC)
    return pl.kernel(
        _make_sc_body(per_w),
        mesh=mesh,
        out_type=jax.ShapeDtypeStruct((n,), jnp.float32),
        scratch_types=[
            pltpu.VMEM((per_w,), jnp.int32),
            pltpu.VMEM((per_w,), jnp.float32),
            pltpu.VMEM((_N_CONES,), jnp.float32),
            pltpu.VMEM((_N_CONES,), jnp.float32),
            pltpu.VMEM((per_w,), jnp.float32),
            pltpu.SemaphoreType.DMA,
            pltpu.SemaphoreType.DMA,
            pltpu.SemaphoreType.DMA,
            pltpu.SemaphoreType.DMA,
        ],
    )


def kernel(pred_rotmats, scores, grid_rotmats, medians, mads):
    grid9 = grid_rotmats.reshape(_N_SO3, 9)
    pred9 = pred_rotmats.reshape(_B, 9)
    cone = _tc_cone_indices(pred9, grid9)
    return _sc_normalize(_B)(cone, scores, medians, mads)
